# baseline (device time: 25503 ns/iter reference)
import jax
import jax.numpy as jnp
from jax import lax
from jax.experimental import pallas as pl
from jax.experimental.pallas import tpu as pltpu

N_DEV = 4
N_LAYERS = 3


def kernel(x, Win0, Wout0, Win1, Wout1, Win2, Wout2):
    b, d_shard = x.shape
    _, h_dim = Win0.shape

    def body(x_ref, win0_ref, wout0_ref, win1_ref, wout1_ref, win2_ref,
             wout2_ref, out_ref, src_ref, comm_ref, send_sems, recv_sems):
        my = lax.axis_index("i")

        barrier_sem = pltpu.get_barrier_semaphore()
        for d in range(1, N_DEV):
            pl.semaphore_signal(
                barrier_sem, inc=1,
                device_id=((my + d) % N_DEV,),
                device_id_type=pl.DeviceIdType.MESH,
            )
        pl.semaphore_wait(barrier_sem, N_DEV - 1)

        wins = [win0_ref, win1_ref, win2_ref]
        wouts = [wout0_ref, wout1_ref, wout2_ref]

        xv = x_ref[:, :]
        for l in range(N_LAYERS):
            partial = jnp.dot(xv, wins[l][:, :],
                              preferred_element_type=jnp.float32)
            src_ref[l, :, :] = partial

            rdmas = []
            for d in range(1, N_DEV):
                rdma = pltpu.make_async_remote_copy(
                    src_ref=src_ref.at[l],
                    dst_ref=comm_ref.at[l, d - 1],
                    send_sem=send_sems.at[l, d - 1],
                    recv_sem=recv_sems.at[l, d - 1],
                    device_id=((my + d) % N_DEV,),
                    device_id_type=pl.DeviceIdType.MESH,
                )
                rdma.start()
                rdmas.append(rdma)
            for rdma in rdmas:
                rdma.wait()

            h = (partial
                 + comm_ref[l, 0, :, :]
                 + comm_ref[l, 1, :, :]
                 + comm_ref[l, 2, :, :])
            h = jnp.maximum(h, 0.0)
            xv = jnp.dot(h, wouts[l][:, :],
                         preferred_element_type=jnp.float32)

        out_ref[:, :] = xv

    return pl.pallas_call(
        body,
        out_shape=jax.ShapeDtypeStruct((b, d_shard), jnp.float32),
        in_specs=[pl.BlockSpec(memory_space=pltpu.VMEM)] * 7,
        out_specs=pl.BlockSpec(memory_space=pltpu.VMEM),
        scratch_shapes=[
            pltpu.VMEM((N_LAYERS, b, h_dim), jnp.float32),
            pltpu.VMEM((N_LAYERS, N_DEV - 1, b, h_dim), jnp.float32),
            pltpu.SemaphoreType.DMA((N_LAYERS, N_DEV - 1)),
            pltpu.SemaphoreType.DMA((N_LAYERS, N_DEV - 1)),
        ],
        compiler_params=pltpu.CompilerParams(collective_id=0),
    )(x, Win0, Wout0, Win1, Wout1, Win2, Wout2)


# device time: 24747 ns/iter; 1.0305x vs baseline; 1.0305x over previous
import jax
import jax.numpy as jnp
from jax import lax
from jax.experimental import pallas as pl
from jax.experimental.pallas import tpu as pltpu

N_DEV = 4
N_LAYERS = 3


def kernel(x, Win0, Wout0, Win1, Wout1, Win2, Wout2):
    b, d_shard = x.shape
    _, h_dim = Win0.shape

    def body(x_ref, win0_ref, wout0_ref, win1_ref, wout1_ref, win2_ref,
             wout2_ref, out_ref, src_ref, comm_ref, send_sems, recv_sems):
        my = lax.axis_index("i")

        barrier_sem = pltpu.get_barrier_semaphore()
        for d in range(1, N_DEV):
            pl.semaphore_signal(
                barrier_sem, inc=1,
                device_id=((my + d) % N_DEV,),
                device_id_type=pl.DeviceIdType.MESH,
            )
        pl.semaphore_wait(barrier_sem, N_DEV - 1)

        wins = [win0_ref, win1_ref, win2_ref]
        wouts = [wout0_ref, wout1_ref, wout2_ref]

        xv = x_ref[:, :]
        all_rdmas = []
        for l in range(N_LAYERS):
            partial = jnp.dot(xv, wins[l][:, :],
                              preferred_element_type=jnp.float32)
            src_ref[l, :, :] = partial

            rdmas = {}
            for d in (2, 1, 3):
                rdma = pltpu.make_async_remote_copy(
                    src_ref=src_ref.at[l],
                    dst_ref=comm_ref.at[l, d - 1],
                    send_sem=send_sems.at[l, d - 1],
                    recv_sem=recv_sems.at[l, d - 1],
                    device_id=((my + d) % N_DEV,),
                    device_id_type=pl.DeviceIdType.MESH,
                )
                rdma.start()
                rdmas[d] = rdma
                all_rdmas.append(rdma)

            acc = partial
            for d in (1, 3, 2):
                rdmas[d].wait_recv()
                acc = acc + comm_ref[l, d - 1, :, :]
            h = jnp.maximum(acc, 0.0)
            xv = jnp.dot(h, wouts[l][:, :],
                         preferred_element_type=jnp.float32)

        out_ref[:, :] = xv
        for rdma in all_rdmas:
            rdma.wait_send()

    return pl.pallas_call(
        body,
        out_shape=jax.ShapeDtypeStruct((b, d_shard), jnp.float32),
        in_specs=[pl.BlockSpec(memory_space=pltpu.VMEM)] * 7,
        out_specs=pl.BlockSpec(memory_space=pltpu.VMEM),
        scratch_shapes=[
            pltpu.VMEM((N_LAYERS, b, h_dim), jnp.float32),
            pltpu.VMEM((N_LAYERS, N_DEV - 1, b, h_dim), jnp.float32),
            pltpu.SemaphoreType.DMA((N_LAYERS, N_DEV - 1)),
            pltpu.SemaphoreType.DMA((N_LAYERS, N_DEV - 1)),
        ],
        compiler_params=pltpu.CompilerParams(collective_id=0),
    )(x, Win0, Wout0, Win1, Wout1, Win2, Wout2)


# device time: 20635 ns/iter; 1.2359x vs baseline; 1.1993x over previous
import jax
import jax.numpy as jnp
from jax import lax
from jax.experimental import pallas as pl
from jax.experimental.pallas import tpu as pltpu

N_DEV = 4
N_LAYERS = 3


def kernel(x, Win0, Wout0, Win1, Wout1, Win2, Wout2):
    b, d_shard = x.shape
    _, h_dim = Win0.shape

    def body(x_ref, win0_ref, wout0_ref, win1_ref, wout1_ref, win2_ref,
             wout2_ref, out_ref, src_ref, comm_ref, send_sems, recv_sems):
        my = lax.axis_index("i")

        barrier_sem = pltpu.get_barrier_semaphore()
        for d in range(1, N_DEV):
            pl.semaphore_signal(
                barrier_sem, inc=1,
                device_id=((my + d) % N_DEV,),
                device_id_type=pl.DeviceIdType.MESH,
            )
        pl.semaphore_wait(barrier_sem, N_DEV - 1)

        wins = [win0_ref, win1_ref, win2_ref]
        wouts = [wout0_ref, wout1_ref, wout2_ref]

        xv = x_ref[:, :]
        all_rdmas = []
        for l in range(N_LAYERS):
            partial = jnp.dot(xv, wins[l][:, :],
                              preferred_element_type=jnp.float32)
            src_ref[l, :, :] = partial.astype(jnp.bfloat16)

            rdmas = {}
            for d in (2, 1, 3):
                rdma = pltpu.make_async_remote_copy(
                    src_ref=src_ref.at[l],
                    dst_ref=comm_ref.at[l, d - 1],
                    send_sem=send_sems.at[l, d - 1],
                    recv_sem=recv_sems.at[l, d - 1],
                    device_id=((my + d) % N_DEV,),
                    device_id_type=pl.DeviceIdType.MESH,
                )
                rdma.start()
                rdmas[d] = rdma
                all_rdmas.append(rdma)

            acc = partial
            for d in (1, 3, 2):
                rdmas[d].wait_recv()
                acc = acc + comm_ref[l, d - 1, :, :].astype(jnp.float32)
            h = jnp.maximum(acc, 0.0)
            xv = jnp.dot(h, wouts[l][:, :],
                         preferred_element_type=jnp.float32)

        out_ref[:, :] = xv
        for rdma in all_rdmas:
            rdma.wait_send()

    return pl.pallas_call(
        body,
        out_shape=jax.ShapeDtypeStruct((b, d_shard), jnp.float32),
        in_specs=[pl.BlockSpec(memory_space=pltpu.VMEM)] * 7,
        out_specs=pl.BlockSpec(memory_space=pltpu.VMEM),
        scratch_shapes=[
            pltpu.VMEM((N_LAYERS, b, h_dim), jnp.bfloat16),
            pltpu.VMEM((N_LAYERS, N_DEV - 1, b, h_dim), jnp.bfloat16),
            pltpu.SemaphoreType.DMA((N_LAYERS, N_DEV - 1)),
            pltpu.SemaphoreType.DMA((N_LAYERS, N_DEV - 1)),
        ],
        compiler_params=pltpu.CompilerParams(collective_id=0),
    )(x, Win0, Wout0, Win1, Wout1, Win2, Wout2)


# device time: 20409 ns/iter; 1.2496x vs baseline; 1.0111x over previous
import jax
import jax.numpy as jnp
from jax import lax
from jax.experimental import pallas as pl
from jax.experimental.pallas import tpu as pltpu

N_DEV = 4
N_LAYERS = 3


def kernel(x, Win0, Wout0, Win1, Wout1, Win2, Wout2):
    b, d_shard = x.shape
    _, h_dim = Win0.shape

    def body(x_ref, win0_ref, wout0_ref, win1_ref, wout1_ref, win2_ref,
             wout2_ref, out_ref, src_ref, comm_ref, send_sems, recv_sems):
        my = lax.axis_index("i")

        barrier_sem = pltpu.get_barrier_semaphore()
        for d in range(1, N_DEV):
            pl.semaphore_signal(
                barrier_sem, inc=1,
                device_id=((my + d) % N_DEV,),
                device_id_type=pl.DeviceIdType.MESH,
            )
        pl.semaphore_wait(barrier_sem, N_DEV - 1)

        wins = [win0_ref, win1_ref, win2_ref]
        wouts = [wout0_ref, wout1_ref, wout2_ref]

        xv = x_ref[:, :].astype(jnp.bfloat16)
        all_rdmas = []
        for l in range(N_LAYERS):
            partial = jnp.dot(xv, wins[l][:, :].astype(jnp.bfloat16),
                              preferred_element_type=jnp.float32)
            src_ref[l, :, :] = partial.astype(jnp.bfloat16)

            rdmas = {}
            for d in (2, 1, 3):
                rdma = pltpu.make_async_remote_copy(
                    src_ref=src_ref.at[l],
                    dst_ref=comm_ref.at[l, d - 1],
                    send_sem=send_sems.at[l, d - 1],
                    recv_sem=recv_sems.at[l, d - 1],
                    device_id=((my + d) % N_DEV,),
                    device_id_type=pl.DeviceIdType.MESH,
                )
                rdma.start()
                rdmas[d] = rdma
                all_rdmas.append(rdma)

            acc = partial
            for d in (1, 3, 2):
                rdmas[d].wait_recv()
                acc = acc + comm_ref[l, d - 1, :, :].astype(jnp.float32)
            h = jnp.maximum(acc, 0.0).astype(jnp.bfloat16)
            xv = jnp.dot(h, wouts[l][:, :].astype(jnp.bfloat16),
                         preferred_element_type=jnp.float32).astype(jnp.bfloat16)

        out_ref[:, :] = xv.astype(jnp.float32)
        for rdma in all_rdmas:
            rdma.wait_send()

    return pl.pallas_call(
        body,
        out_shape=jax.ShapeDtypeStruct((b, d_shard), jnp.float32),
        in_specs=[pl.BlockSpec(memory_space=pltpu.VMEM)] * 7,
        out_specs=pl.BlockSpec(memory_space=pltpu.VMEM),
        scratch_shapes=[
            pltpu.VMEM((N_LAYERS, b, h_dim), jnp.bfloat16),
            pltpu.VMEM((N_LAYERS, N_DEV - 1, b, h_dim), jnp.bfloat16),
            pltpu.SemaphoreType.DMA((N_LAYERS, N_DEV - 1)),
            pltpu.SemaphoreType.DMA((N_LAYERS, N_DEV - 1)),
        ],
        compiler_params=pltpu.CompilerParams(collective_id=0),
    )(x, Win0, Wout0, Win1, Wout1, Win2, Wout2)
